# Initial kernel scaffold; baseline (speedup 1.0000x reference)
import functools
import jax
import jax.numpy as jnp
from jax import lax
from jax.experimental import pallas as pl
from jax.experimental.pallas import tpu as pltpu
from jax.experimental.pallas import tpu_sc as plsc

NUM_NODES = 2708
NUM_EDGES = 500000
D = 128
NP = 2816
NC = 2
NS = 16
NW = NC * NS
N_CHUNKS = 4
CHUNK_ROWS = NP // N_CHUNKS
ROWS_PER_SUB = CHUNK_ROWS // NS
PASSES = N_CHUNKS // NC
EB = 128
EPW = 15744
E_PAD = EPW * NW
NBLK = EPW // EB


def _hist_body(src_hbm, dst_hbm, zeros_hbm, a_hbm, src_v, dst_v, idx_v, val_v):
    c = lax.axis_index("c")
    s = lax.axis_index("s")
    wid = c * NS + s
    base = wid * EPW
    pltpu.sync_copy(src_hbm.at[pl.ds(base, EPW)], src_v)
    pltpu.sync_copy(dst_hbm.at[pl.ds(base, EPW)], dst_v)

    def one_pass(p, chunk_sh):
        ck = p * NC + c
        lo = ck * CHUNK_ROWS
        slab = s * ROWS_PER_SUB * NP
        pltpu.sync_copy(zeros_hbm, chunk_sh.at[pl.ds(slab, ROWS_PER_SUB * NP)])
        plsc.subcore_barrier()

        def blk(j, carry):
            off = j * EB
            for i in range(EB // 16):
                d = dst_v[pl.ds(off + i * 16, 16)]
                sv = src_v[pl.ds(off + i * 16, 16)]
                m = (d >= lo) & (d < lo + CHUNK_ROWS)
                fl = jnp.where(m, (d - lo) * NP + sv, 0)
                v = jnp.where(m, 1.0, 0.0).astype(jnp.float32)
                idx_v[pl.ds(i * 16, 16)] = fl
                val_v[pl.ds(i * 16, 16)] = v
            pltpu.sync_copy(val_v, chunk_sh.at[idx_v], add=True)
            return carry

        lax.fori_loop(0, NBLK, blk, 0)
        plsc.subcore_barrier()
        pltpu.sync_copy(
            chunk_sh.at[pl.ds(slab, ROWS_PER_SUB * NP)],
            a_hbm.at[pl.ds(lo * NP + slab, ROWS_PER_SUB * NP)],
        )
        plsc.subcore_barrier()

    for p in range(PASSES):
        pl.run_scoped(
            functools.partial(one_pass, p),
            pltpu.VMEM_SHARED((CHUNK_ROWS * NP,), jnp.float32),
        )


def _build_counts(src, dst, zeros):
    mesh = plsc.VectorSubcoreMesh(core_axis_name="c", subcore_axis_name="s")
    return pl.kernel(
        _hist_body,
        out_type=jax.ShapeDtypeStruct((NP * NP,), jnp.float32),
        mesh=mesh,
        scratch_types=[
            pltpu.VMEM((EPW,), jnp.int32),
            pltpu.VMEM((EPW,), jnp.int32),
            pltpu.VMEM((EB,), jnp.int32),
            pltpu.VMEM((EB,), jnp.float32),
        ],
    )(src, dst, zeros)


def _mm_body(a_ref, x_ref, w_ref, b_ref, o_ref):
    a = a_ref[...]
    ax = jnp.dot(a, x_ref[...], preferred_element_type=jnp.float32)
    h = lax.dot_general(ax, w_ref[...], (((1,), (1,)), ((), ())),
                        preferred_element_type=jnp.float32)
    deg = jnp.sum(a, axis=1, keepdims=True)
    o_ref[...] = h + deg * b_ref[...]


def _gcn_matmul(a2d, x_pad, weight, bias2d):
    BM = 256
    grid = (NP // BM,)
    return pl.pallas_call(
        _mm_body,
        grid=grid,
        in_specs=[
            pl.BlockSpec((BM, NP), lambda i: (i, 0)),
            pl.BlockSpec((NP, D), lambda i: (0, 0)),
            pl.BlockSpec((D, D), lambda i: (0, 0)),
            pl.BlockSpec((1, D), lambda i: (0, 0)),
        ],
        out_specs=pl.BlockSpec((BM, D), lambda i: (i, 0)),
        out_shape=jax.ShapeDtypeStruct((NP, D), jnp.float32),
    )(a2d, x_pad, weight, bias2d)


def kernel(x, edge_index, weight, bias):
    src = edge_index[0].astype(jnp.int32)
    dst = edge_index[1].astype(jnp.int32)
    pad = E_PAD - NUM_EDGES
    src = jnp.concatenate([src, jnp.zeros((pad,), jnp.int32)])
    dst = jnp.concatenate([dst, jnp.full((pad,), NP * 2, jnp.int32)])
    zeros = jnp.zeros((ROWS_PER_SUB * NP,), jnp.float32)

    a_flat = _build_counts(src, dst, zeros)
    a2d = a_flat.reshape(NP, NP)

    x_pad = jnp.zeros((NP, D), jnp.float32).at[:NUM_NODES].set(x)
    out = _gcn_matmul(a2d, x_pad, weight, bias.reshape(1, D))
    return out[:NUM_NODES]


# speedup vs baseline: 1.2895x; 1.2895x over previous
import functools
import jax
import jax.numpy as jnp
from jax import lax
from jax.experimental import pallas as pl
from jax.experimental.pallas import tpu as pltpu
from jax.experimental.pallas import tpu_sc as plsc

NUM_NODES = 2708
NUM_EDGES = 500000
D = 128
NP = 2816
NC = 2
NS = 16
NW = NC * NS
N_CHUNKS = 8
CHUNK_ROWS = NP // N_CHUNKS
ROWS_PER_SUB = CHUNK_ROWS // NS
PASSES = N_CHUNKS // NC
EB = 128
E_PAD = 503808
EPS = E_PAD // NS
NBLK = EPS // EB


ZW = NP
NZ = ROWS_PER_SUB


def _hist_body(src_hbm, dst_hbm, a_hbm, src_v, dst_v, idx_v, val_v, zbuf,
               chunk_sh):
    c = lax.axis_index("c")
    s = lax.axis_index("s")
    base = s * EPS
    pltpu.sync_copy(src_hbm.at[pl.ds(base, EPS)], src_v)
    pltpu.sync_copy(dst_hbm.at[pl.ds(base, EPS)], dst_v)

    def zfill(i, carry):
        zbuf[pl.ds(i * 16, 16)] = jnp.zeros((16,), jnp.float32)
        return carry

    lax.fori_loop(0, ZW // 16, zfill, 0)

    def one_pass(p):
        ck = p * NC + c
        lo = ck * CHUNK_ROWS
        slab = s * ROWS_PER_SUB * NP
        for z in range(NZ):
            pltpu.sync_copy(zbuf, chunk_sh.at[pl.ds(slab + z * ZW, ZW)])
        plsc.subcore_barrier()

        def blk(j, carry):
            off = j * EB
            for i in range(EB // 16):
                d = dst_v[pl.ds(off + i * 16, 16)]
                sv = src_v[pl.ds(off + i * 16, 16)]
                m = (d >= lo) & (d < lo + CHUNK_ROWS)
                fl = jnp.where(m, (d - lo) * NP + sv, 0)
                v = jnp.where(m, 1.0, 0.0).astype(jnp.float32)
                idx_v[pl.ds(i * 16, 16)] = fl
                val_v[pl.ds(i * 16, 16)] = v
            pltpu.sync_copy(val_v, chunk_sh.at[idx_v], add=True)
            return carry

        lax.fori_loop(0, NBLK, blk, 0)
        plsc.subcore_barrier()
        pltpu.sync_copy(
            chunk_sh.at[pl.ds(slab, ROWS_PER_SUB * NP)],
            a_hbm.at[pl.ds(lo * NP + slab, ROWS_PER_SUB * NP)],
        )
        plsc.subcore_barrier()

    for p in range(PASSES):
        one_pass(p)


def _build_counts(src, dst):
    mesh = plsc.VectorSubcoreMesh(core_axis_name="c", subcore_axis_name="s")
    return pl.kernel(
        _hist_body,
        out_type=jax.ShapeDtypeStruct((NP * NP,), jnp.float32),
        mesh=mesh,
        scratch_types=[
            pltpu.VMEM((EPS,), jnp.int32),
            pltpu.VMEM((EPS,), jnp.int32),
            pltpu.VMEM((EB,), jnp.int32),
            pltpu.VMEM((EB,), jnp.float32),
            pltpu.VMEM((ZW,), jnp.float32),
            pltpu.VMEM_SHARED((CHUNK_ROWS * NP,), jnp.float32),
        ],
    )(src, dst)


def _mm_body(a_ref, x_ref, w_ref, b_ref, o_ref):
    a = a_ref[...]
    ax = jnp.dot(a, x_ref[...], preferred_element_type=jnp.float32)
    h = lax.dot_general(ax, w_ref[...], (((1,), (1,)), ((), ())),
                        preferred_element_type=jnp.float32)
    deg = jnp.sum(a, axis=1, keepdims=True)
    o_ref[...] = h + deg * b_ref[...]


def _gcn_matmul(a2d, x_pad, weight, bias2d):
    BM = 256
    grid = (NP // BM,)
    return pl.pallas_call(
        _mm_body,
        grid=grid,
        in_specs=[
            pl.BlockSpec((BM, NP), lambda i: (i, 0)),
            pl.BlockSpec((NP, D), lambda i: (0, 0)),
            pl.BlockSpec((D, D), lambda i: (0, 0)),
            pl.BlockSpec((1, D), lambda i: (0, 0)),
        ],
        out_specs=pl.BlockSpec((BM, D), lambda i: (i, 0)),
        out_shape=jax.ShapeDtypeStruct((NP, D), jnp.float32),
    )(a2d, x_pad, weight, bias2d)


def kernel(x, edge_index, weight, bias):
    src = edge_index[0].astype(jnp.int32)
    dst = edge_index[1].astype(jnp.int32)
    pad = E_PAD - NUM_EDGES
    src = jnp.concatenate([src, jnp.zeros((pad,), jnp.int32)])
    dst = jnp.concatenate([dst, jnp.full((pad,), NP * 2, jnp.int32)])

    a_flat = _build_counts(src, dst)
    a2d = a_flat.reshape(NP, NP)

    x_pad = jnp.zeros((NP, D), jnp.float32).at[:NUM_NODES].set(x)
    out = _gcn_matmul(a2d, x_pad, weight, bias.reshape(1, D))
    return out[:NUM_NODES]
